# baseline (speedup 1.0000x reference)
import functools

import jax
import jax.numpy as jnp
from jax import lax
from jax.experimental import pallas as pl
from jax.experimental.pallas import tpu as pltpu
from jax.experimental.pallas import tpu_sc as plsc

_N = 10000
_E = 320000
_D = 128
_C = 64
_H = _D // 2

_NC = 2
_NS = 16
_CH = 64
_NCH = 313
_EPT = _NCH * _CH
_EPAD = _NS * _EPT - _E
_NA = _N + 8
_STRIPE = 624
_TAIL = _NA - _STRIPE * _NS


def _sc_agg_body(with_deg, *refs):
    if with_deg:
        (srcp, dstp, table, z2, out, dout,
         sidx, didx, rows_a, rows_b, acc, sem_a, sem_b,
         ones, dacc, dzero) = refs
    else:
        (srcp, dstp, table, z2, out,
         sidx, didx, rows_a, rows_b, acc, sem_a, sem_b) = refs
    c = lax.axis_index("c")
    s = lax.axis_index("s")

    pltpu.sync_copy(z2.at[pl.ds(s * _STRIPE, _STRIPE)],
                    acc.at[pl.ds(s * _STRIPE, _STRIPE)])

    @pl.when(s == _NS - 1)
    def _():
        pltpu.sync_copy(z2.at[pl.ds(_STRIPE * _NS, _TAIL)],
                        acc.at[pl.ds(_STRIPE * _NS, _TAIL)])

    if with_deg:
        for i in range(_STRIPE // 16):
            dzero[pl.ds(i * 16, 16)] = jnp.zeros((16,), jnp.float32)
        pltpu.sync_copy(dzero, dacc.at[pl.ds(s * _STRIPE, _STRIPE)])

        @pl.when(s == _NS - 1)
        def _():
            pltpu.sync_copy(dzero.at[pl.ds(0, _TAIL)],
                            dacc.at[pl.ds(_STRIPE * _NS, _TAIL)])

        for i in range(_CH // 16):
            ones[pl.ds(i * 16, 16)] = jnp.ones((16,), jnp.float32)

    pltpu.sync_copy(srcp.at[s], sidx)
    pltpu.sync_copy(dstp.at[s], didx)

    plsc.subcore_barrier()

    def gstart(k, buf, sem):
        pltpu.async_copy(table.at[c].at[sidx.at[k]], buf, sem)

    def gwait(k, buf, sem):
        pltpu.make_async_copy(table.at[c].at[sidx.at[k]], buf, sem).wait()

    def scat(k, buf):
        pltpu.sync_copy(buf, acc.at[didx.at[k]], add=True)
        if with_deg:
            pltpu.sync_copy(ones, dacc.at[didx.at[k]], add=True)

    gstart(0, rows_a, sem_a)

    def step(i, carry):
        ka = 2 * i
        kb = 2 * i + 1
        gstart(kb, rows_b, sem_b)
        gwait(ka, rows_a, sem_a)
        scat(ka, rows_a)
        gstart(ka + 2, rows_a, sem_a)
        gwait(kb, rows_b, sem_b)
        scat(kb, rows_b)
        return carry

    lax.fori_loop(0, (_NCH - 1) // 2, step, 0)
    gwait(_NCH - 1, rows_a, sem_a)
    scat(_NCH - 1, rows_a)
    plsc.subcore_barrier()

    pltpu.sync_copy(acc.at[pl.ds(s * _STRIPE, _STRIPE)],
                    out.at[c, pl.ds(s * _STRIPE, _STRIPE)])

    @pl.when(s == _NS - 1)
    def _():
        pltpu.sync_copy(acc.at[pl.ds(_STRIPE * _NS, _N - _STRIPE * _NS)],
                        out.at[c, pl.ds(_STRIPE * _NS, _N - _STRIPE * _NS)])

    if with_deg:
        @pl.when(c == 0)
        def _():
            pltpu.sync_copy(dacc.at[pl.ds(s * _STRIPE, _STRIPE)], dzero)
            pltpu.sync_copy(dzero, dout.at[pl.ds(s * _STRIPE, _STRIPE)])

        @pl.when(jnp.logical_and(c == 0, s == _NS - 1))
        def _():
            nt = _N - _STRIPE * _NS
            pltpu.sync_copy(dacc.at[pl.ds(_STRIPE * _NS, nt)],
                            dzero.at[pl.ds(0, nt)])
            pltpu.sync_copy(dzero.at[pl.ds(0, nt)],
                            dout.at[pl.ds(_STRIPE * _NS, nt)])


def _make_sc_agg(with_deg):
    mesh = plsc.VectorSubcoreMesh(core_axis_name="c", subcore_axis_name="s")
    out_type = [jax.ShapeDtypeStruct((_NC, _N, _H), jnp.bfloat16)]
    scratch = [
        pltpu.VMEM((_NCH, _CH), jnp.int32),
        pltpu.VMEM((_NCH, _CH), jnp.int32),
        pltpu.VMEM((_CH, _H), jnp.bfloat16),
        pltpu.VMEM((_CH, _H), jnp.bfloat16),
        pltpu.VMEM_SHARED((_NA, _H), jnp.bfloat16),
        pltpu.SemaphoreType.DMA,
        pltpu.SemaphoreType.DMA,
    ]
    if with_deg:
        out_type.append(jax.ShapeDtypeStruct((_N,), jnp.float32))
        scratch += [
            pltpu.VMEM((_CH,), jnp.float32),
            pltpu.VMEM_SHARED((_NA,), jnp.float32),
            pltpu.VMEM((_STRIPE,), jnp.float32),
        ]
    return pl.kernel(
        functools.partial(_sc_agg_body, with_deg),
        out_type=out_type,
        mesh=mesh,
        scratch_types=scratch,
        compiler_params=pltpu.CompilerParams(use_tc_tiling_on_sc=False),
    )


_sc_agg_deg = _make_sc_agg(True)
_sc_agg = _make_sc_agg(False)


_RB = 1000
_GRID = _N // _RB


def _tc_layer0_body(x_ref, ws_ref, wn_ref, b_ref, zs_ref, zn_ref):
    x = x_ref[...]
    zs_ref[...] = (jnp.dot(x, ws_ref[...], preferred_element_type=jnp.float32)
                   + b_ref[...])
    zn = jnp.dot(x, wn_ref[...],
                 preferred_element_type=jnp.float32).astype(jnp.bfloat16)
    zn_ref[0] = zn[:, :_H]
    zn_ref[1] = zn[:, _H:]


def _tc_layer_body(zs_ref, a_ref, d_ref, ws_ref, wn_ref, b_ref,
                   ozs_ref, ozn_ref):
    deg = jnp.maximum(d_ref[...], 1.0)
    agg = jnp.concatenate([a_ref[0], a_ref[1]],
                          axis=1).astype(jnp.float32) / deg
    h = jnp.maximum(zs_ref[...] + agg, 0.0)
    ozs_ref[...] = (jnp.dot(h, ws_ref[...], preferred_element_type=jnp.float32)
                    + b_ref[...])
    zn = jnp.dot(h, wn_ref[...],
                 preferred_element_type=jnp.float32).astype(jnp.bfloat16)
    ozn_ref[0] = zn[:, :_H]
    ozn_ref[1] = zn[:, _H:]


def _tc_final_body(zs_ref, a_ref, d_ref, wf_ref, bf_ref, o_ref):
    i = pl.program_id(0)
    deg = jnp.maximum(d_ref[...], 1.0)
    agg = jnp.concatenate([a_ref[0], a_ref[1]],
                          axis=1).astype(jnp.float32) / deg
    h = jnp.maximum(zs_ref[...] + agg, 0.0)
    part = jnp.sum(h, axis=0, keepdims=True) * (1.0 / _N)
    part = jnp.dot(part, wf_ref[...], preferred_element_type=jnp.float32)

    @pl.when(i == 0)
    def _():
        o_ref[...] = bf_ref[...]

    o_ref[...] += part


_row_spec = pl.BlockSpec((_RB, _D), lambda i: (i, 0))
_a_spec = pl.BlockSpec((_NC, _RB, _H), lambda i: (0, i, 0))
_d_spec = pl.BlockSpec((_RB, 1), lambda i: (i, 0))
_w_spec = pl.BlockSpec((_D, _D), lambda i: (0, 0))
_b_spec = pl.BlockSpec((1, _D), lambda i: (0, 0))

_tc_layer0 = pl.pallas_call(
    _tc_layer0_body,
    grid=(_GRID,),
    in_specs=[_row_spec, _w_spec, _w_spec, _b_spec],
    out_specs=[_row_spec, _a_spec],
    out_shape=[jax.ShapeDtypeStruct((_N, _D), jnp.float32),
               jax.ShapeDtypeStruct((_NC, _N, _H), jnp.bfloat16)],
)

_tc_layer = pl.pallas_call(
    _tc_layer_body,
    grid=(_GRID,),
    in_specs=[_row_spec, _a_spec, _d_spec, _w_spec, _w_spec, _b_spec],
    out_specs=[_row_spec, _a_spec],
    out_shape=[jax.ShapeDtypeStruct((_N, _D), jnp.float32),
               jax.ShapeDtypeStruct((_NC, _N, _H), jnp.bfloat16)],
)

_tc_final = pl.pallas_call(
    _tc_final_body,
    grid=(_GRID,),
    in_specs=[_row_spec, _a_spec, _d_spec,
              pl.BlockSpec((_D, _C), lambda i: (0, 0)),
              pl.BlockSpec((1, _C), lambda i: (0, 0))],
    out_specs=pl.BlockSpec((1, _C), lambda i: (0, 0)),
    out_shape=jax.ShapeDtypeStruct((1, _C), jnp.float32),
)


def kernel(x, edge_index, ws0, wn0, b0, ws1, wn1, b1, ws2, wn2, b2, wf, bf):
    src = jnp.concatenate(
        [edge_index[0], jnp.zeros((_EPAD,), jnp.int32)]).reshape(
            _NS, _NCH, _CH)
    dst = jnp.concatenate(
        [edge_index[1], jnp.full((_EPAD,), _N, jnp.int32)]).reshape(
            _NS, _NCH, _CH)
    z2 = jnp.zeros((_NA, _H), jnp.bfloat16)

    zs0, zn0 = _tc_layer0(x, ws0, wn0, b0.reshape(1, _D))
    a0, dvec = _sc_agg_deg(src, dst, zn0, z2)
    d = dvec.reshape(_N, 1)
    zs1, zn1 = _tc_layer(zs0, a0, d, ws1, wn1, b1.reshape(1, _D))
    (a1,) = _sc_agg(src, dst, zn1, z2)
    zs2, zn2 = _tc_layer(zs1, a1, d, ws2, wn2, b2.reshape(1, _D))
    (a2,) = _sc_agg(src, dst, zn2, z2)
    out = _tc_final(zs2, a2, d, wf, bf.reshape(1, _C))
    return out


# speedup vs baseline: 1.2577x; 1.2577x over previous
import functools

import jax
import jax.numpy as jnp
from jax import lax
from jax.experimental import pallas as pl
from jax.experimental.pallas import tpu as pltpu
from jax.experimental.pallas import tpu_sc as plsc

_N = 10000
_E = 320000
_D = 128
_C = 64
_H = _D // 2

_NC = 2
_NS = 16
_CH = 128
_NCH = 157
_EPT = _NCH * _CH
_EPAD = _NS * _EPT - _E
_NA = _N + 8
_STRIPE = 624
_TAIL = _NA - _STRIPE * _NS


def _sc_agg_body(with_deg, *refs):
    if with_deg:
        (srcp, dstp, table, z2, out, dout,
         sidx, didx, rows_a, rows_b, acc, sem_a, sem_b,
         ones, dacc, dzero) = refs
    else:
        (srcp, dstp, table, z2, out,
         sidx, didx, rows_a, rows_b, acc, sem_a, sem_b) = refs
    c = lax.axis_index("c")
    s = lax.axis_index("s")

    pltpu.sync_copy(z2.at[pl.ds(s * _STRIPE, _STRIPE)],
                    acc.at[pl.ds(s * _STRIPE, _STRIPE)])

    @pl.when(s == _NS - 1)
    def _():
        pltpu.sync_copy(z2.at[pl.ds(_STRIPE * _NS, _TAIL)],
                        acc.at[pl.ds(_STRIPE * _NS, _TAIL)])

    if with_deg:
        for i in range(_STRIPE // 16):
            dzero[pl.ds(i * 16, 16)] = jnp.zeros((16,), jnp.float32)
        pltpu.sync_copy(dzero, dacc.at[pl.ds(s * _STRIPE, _STRIPE)])

        @pl.when(s == _NS - 1)
        def _():
            pltpu.sync_copy(dzero.at[pl.ds(0, _TAIL)],
                            dacc.at[pl.ds(_STRIPE * _NS, _TAIL)])

        for i in range(_CH // 16):
            ones[pl.ds(i * 16, 16)] = jnp.ones((16,), jnp.float32)

    pltpu.sync_copy(srcp.at[s], sidx)
    pltpu.sync_copy(dstp.at[s], didx)

    plsc.subcore_barrier()

    def gstart(k, buf, sem):
        pltpu.async_copy(table.at[c].at[sidx.at[k]], buf, sem)

    def gwait(k, buf, sem):
        pltpu.make_async_copy(table.at[c].at[sidx.at[k]], buf, sem).wait()

    def scat(k, buf):
        pltpu.sync_copy(buf, acc.at[didx.at[k]], add=True)
        if with_deg:
            pltpu.sync_copy(ones, dacc.at[didx.at[k]], add=True)

    gstart(0, rows_a, sem_a)

    def step(i, carry):
        ka = 2 * i
        kb = 2 * i + 1
        gstart(kb, rows_b, sem_b)
        gwait(ka, rows_a, sem_a)
        scat(ka, rows_a)
        gstart(ka + 2, rows_a, sem_a)
        gwait(kb, rows_b, sem_b)
        scat(kb, rows_b)
        return carry

    lax.fori_loop(0, (_NCH - 1) // 2, step, 0)
    gwait(_NCH - 1, rows_a, sem_a)
    scat(_NCH - 1, rows_a)
    plsc.subcore_barrier()

    pltpu.sync_copy(acc.at[pl.ds(s * _STRIPE, _STRIPE)],
                    out.at[c, pl.ds(s * _STRIPE, _STRIPE)])

    @pl.when(s == _NS - 1)
    def _():
        pltpu.sync_copy(acc.at[pl.ds(_STRIPE * _NS, _N - _STRIPE * _NS)],
                        out.at[c, pl.ds(_STRIPE * _NS, _N - _STRIPE * _NS)])

    if with_deg:
        @pl.when(c == 0)
        def _():
            pltpu.sync_copy(dacc.at[pl.ds(s * _STRIPE, _STRIPE)], dzero)
            pltpu.sync_copy(dzero, dout.at[pl.ds(s * _STRIPE, _STRIPE)])

        @pl.when(jnp.logical_and(c == 0, s == _NS - 1))
        def _():
            nt = _N - _STRIPE * _NS
            pltpu.sync_copy(dacc.at[pl.ds(_STRIPE * _NS, nt)],
                            dzero.at[pl.ds(0, nt)])
            pltpu.sync_copy(dzero.at[pl.ds(0, nt)],
                            dout.at[pl.ds(_STRIPE * _NS, nt)])


def _make_sc_agg(with_deg):
    mesh = plsc.VectorSubcoreMesh(core_axis_name="c", subcore_axis_name="s")
    out_type = [jax.ShapeDtypeStruct((_NC, _N, _H), jnp.bfloat16)]
    scratch = [
        pltpu.VMEM((_NCH, _CH), jnp.int32),
        pltpu.VMEM((_NCH, _CH), jnp.int32),
        pltpu.VMEM((_CH, _H), jnp.bfloat16),
        pltpu.VMEM((_CH, _H), jnp.bfloat16),
        pltpu.VMEM_SHARED((_NA, _H), jnp.bfloat16),
        pltpu.SemaphoreType.DMA,
        pltpu.SemaphoreType.DMA,
    ]
    if with_deg:
        out_type.append(jax.ShapeDtypeStruct((_N,), jnp.float32))
        scratch += [
            pltpu.VMEM((_CH,), jnp.float32),
            pltpu.VMEM_SHARED((_NA,), jnp.float32),
            pltpu.VMEM((_STRIPE,), jnp.float32),
        ]
    return pl.kernel(
        functools.partial(_sc_agg_body, with_deg),
        out_type=out_type,
        mesh=mesh,
        scratch_types=scratch,
        compiler_params=pltpu.CompilerParams(use_tc_tiling_on_sc=False),
    )


_sc_agg_deg = _make_sc_agg(True)
_sc_agg = _make_sc_agg(False)


_RB = 1000
_GRID = _N // _RB


def _tc_layer0_body(x_ref, ws_ref, wn_ref, b_ref, zs_ref, zn_ref):
    x = x_ref[...]
    zs_ref[...] = (jnp.dot(x, ws_ref[...], preferred_element_type=jnp.float32)
                   + b_ref[...])
    zn = jnp.dot(x, wn_ref[...],
                 preferred_element_type=jnp.float32).astype(jnp.bfloat16)
    zn_ref[0] = zn[:, :_H]
    zn_ref[1] = zn[:, _H:]


def _tc_layer_body(zs_ref, a_ref, d_ref, ws_ref, wn_ref, b_ref,
                   ozs_ref, ozn_ref):
    deg = jnp.maximum(d_ref[...], 1.0)
    agg = jnp.concatenate([a_ref[0], a_ref[1]],
                          axis=1).astype(jnp.float32) / deg
    h = jnp.maximum(zs_ref[...] + agg, 0.0)
    ozs_ref[...] = (jnp.dot(h, ws_ref[...], preferred_element_type=jnp.float32)
                    + b_ref[...])
    zn = jnp.dot(h, wn_ref[...],
                 preferred_element_type=jnp.float32).astype(jnp.bfloat16)
    ozn_ref[0] = zn[:, :_H]
    ozn_ref[1] = zn[:, _H:]


def _tc_final_body(zs_ref, a_ref, d_ref, wf_ref, bf_ref, o_ref):
    i = pl.program_id(0)
    deg = jnp.maximum(d_ref[...], 1.0)
    agg = jnp.concatenate([a_ref[0], a_ref[1]],
                          axis=1).astype(jnp.float32) / deg
    h = jnp.maximum(zs_ref[...] + agg, 0.0)
    part = jnp.sum(h, axis=0, keepdims=True) * (1.0 / _N)
    part = jnp.dot(part, wf_ref[...], preferred_element_type=jnp.float32)

    @pl.when(i == 0)
    def _():
        o_ref[...] = bf_ref[...]

    o_ref[...] += part


_row_spec = pl.BlockSpec((_RB, _D), lambda i: (i, 0))
_a_spec = pl.BlockSpec((_NC, _RB, _H), lambda i: (0, i, 0))
_d_spec = pl.BlockSpec((_RB, 1), lambda i: (i, 0))
_w_spec = pl.BlockSpec((_D, _D), lambda i: (0, 0))
_b_spec = pl.BlockSpec((1, _D), lambda i: (0, 0))

_tc_layer0 = pl.pallas_call(
    _tc_layer0_body,
    grid=(_GRID,),
    in_specs=[_row_spec, _w_spec, _w_spec, _b_spec],
    out_specs=[_row_spec, _a_spec],
    out_shape=[jax.ShapeDtypeStruct((_N, _D), jnp.float32),
               jax.ShapeDtypeStruct((_NC, _N, _H), jnp.bfloat16)],
)

_tc_layer = pl.pallas_call(
    _tc_layer_body,
    grid=(_GRID,),
    in_specs=[_row_spec, _a_spec, _d_spec, _w_spec, _w_spec, _b_spec],
    out_specs=[_row_spec, _a_spec],
    out_shape=[jax.ShapeDtypeStruct((_N, _D), jnp.float32),
               jax.ShapeDtypeStruct((_NC, _N, _H), jnp.bfloat16)],
)

_tc_final = pl.pallas_call(
    _tc_final_body,
    grid=(_GRID,),
    in_specs=[_row_spec, _a_spec, _d_spec,
              pl.BlockSpec((_D, _C), lambda i: (0, 0)),
              pl.BlockSpec((1, _C), lambda i: (0, 0))],
    out_specs=pl.BlockSpec((1, _C), lambda i: (0, 0)),
    out_shape=jax.ShapeDtypeStruct((1, _C), jnp.float32),
)


def kernel(x, edge_index, ws0, wn0, b0, ws1, wn1, b1, ws2, wn2, b2, wf, bf):
    src = jnp.concatenate(
        [edge_index[0], jnp.zeros((_EPAD,), jnp.int32)]).reshape(
            _NS, _NCH, _CH)
    dst = jnp.concatenate(
        [edge_index[1], jnp.full((_EPAD,), _N, jnp.int32)]).reshape(
            _NS, _NCH, _CH)
    z2 = jnp.zeros((_NA, _H), jnp.bfloat16)

    zs0, zn0 = _tc_layer0(x, ws0, wn0, b0.reshape(1, _D))
    a0, dvec = _sc_agg_deg(src, dst, zn0, z2)
    d = dvec.reshape(_N, 1)
    zs1, zn1 = _tc_layer(zs0, a0, d, ws1, wn1, b1.reshape(1, _D))
    (a1,) = _sc_agg(src, dst, zn1, z2)
    zs2, zn2 = _tc_layer(zs1, a1, d, ws2, wn2, b2.reshape(1, _D))
    (a2,) = _sc_agg(src, dst, zn2, z2)
    out = _tc_final(zs2, a2, d, wf, bf.reshape(1, _C))
    return out


# speedup vs baseline: 1.2833x; 1.0204x over previous
import functools

import jax
import jax.numpy as jnp
from jax import lax
from jax.experimental import pallas as pl
from jax.experimental.pallas import tpu as pltpu
from jax.experimental.pallas import tpu_sc as plsc

_N = 10000
_E = 320000
_D = 128
_C = 64
_H = _D // 2

_NC = 2
_NS = 16
_CH = 128
_NCH = 157
_EPT = _NCH * _CH
_EPAD = _NS * _EPT - _E
_NA = _N + 8
_STRIPE = 624
_TAIL = _NA - _STRIPE * _NS


def _sc_agg_body(with_deg, *refs):
    if with_deg:
        (srcp, dstp, table, z2, out, dout,
         sidx, didx, rows_a, rows_b, acc, sem_a, sem_b,
         ones, dacc, dzero) = refs
    else:
        (srcp, dstp, table, z2, out,
         sidx, didx, rows_a, rows_b, acc, sem_a, sem_b) = refs
    c = lax.axis_index("c")
    s = lax.axis_index("s")

    pltpu.sync_copy(z2.at[pl.ds(s * _STRIPE, _STRIPE)],
                    acc.at[pl.ds(s * _STRIPE, _STRIPE)])

    @pl.when(s == _NS - 1)
    def _():
        pltpu.sync_copy(z2.at[pl.ds(_STRIPE * _NS, _TAIL)],
                        acc.at[pl.ds(_STRIPE * _NS, _TAIL)])

    if with_deg:
        for i in range(_STRIPE // 16):
            dzero[pl.ds(i * 16, 16)] = jnp.zeros((16,), jnp.float32)
        pltpu.sync_copy(dzero, dacc.at[pl.ds(s * _STRIPE, _STRIPE)])

        @pl.when(s == _NS - 1)
        def _():
            pltpu.sync_copy(dzero.at[pl.ds(0, _TAIL)],
                            dacc.at[pl.ds(_STRIPE * _NS, _TAIL)])

        for i in range(_CH // 16):
            ones[pl.ds(i * 16, 16)] = jnp.ones((16,), jnp.float32)

    pltpu.sync_copy(srcp.at[s], sidx)
    pltpu.sync_copy(dstp.at[s], didx)

    plsc.subcore_barrier()

    def gstart(k, buf, sem):
        pltpu.async_copy(table.at[c].at[sidx.at[k]], buf, sem)

    def gwait(k, buf, sem):
        pltpu.make_async_copy(table.at[c].at[sidx.at[k]], buf, sem).wait()

    def scat(k, buf):
        pltpu.sync_copy(buf, acc.at[didx.at[k]], add=True)
        if with_deg:
            pltpu.sync_copy(ones, dacc.at[didx.at[k]], add=True)

    gstart(0, rows_a, sem_a)

    def step(i, carry):
        ka = 2 * i
        kb = 2 * i + 1
        gstart(kb, rows_b, sem_b)
        gwait(ka, rows_a, sem_a)
        scat(ka, rows_a)
        gstart(ka + 2, rows_a, sem_a)
        gwait(kb, rows_b, sem_b)
        scat(kb, rows_b)
        return carry

    lax.fori_loop(0, (_NCH - 1) // 2, step, 0)
    gwait(_NCH - 1, rows_a, sem_a)
    scat(_NCH - 1, rows_a)
    plsc.subcore_barrier()

    pltpu.sync_copy(acc.at[pl.ds(s * _STRIPE, _STRIPE)],
                    out.at[c, pl.ds(s * _STRIPE, _STRIPE)])

    @pl.when(s == _NS - 1)
    def _():
        pltpu.sync_copy(acc.at[pl.ds(_STRIPE * _NS, _N - _STRIPE * _NS)],
                        out.at[c, pl.ds(_STRIPE * _NS, _N - _STRIPE * _NS)])

    if with_deg:
        @pl.when(c == 0)
        def _():
            pltpu.sync_copy(dacc.at[pl.ds(s * _STRIPE, _STRIPE)], dzero)
            pltpu.sync_copy(dzero, dout.at[pl.ds(s * _STRIPE, _STRIPE)])

        @pl.when(jnp.logical_and(c == 0, s == _NS - 1))
        def _():
            nt = _N - _STRIPE * _NS
            pltpu.sync_copy(dacc.at[pl.ds(_STRIPE * _NS, nt)],
                            dzero.at[pl.ds(0, nt)])
            pltpu.sync_copy(dzero.at[pl.ds(0, nt)],
                            dout.at[pl.ds(_STRIPE * _NS, nt)])


def _make_sc_agg(with_deg):
    mesh = plsc.VectorSubcoreMesh(core_axis_name="c", subcore_axis_name="s")
    out_type = [jax.ShapeDtypeStruct((_NC, _N, _H), jnp.bfloat16)]
    scratch = [
        pltpu.VMEM((_NCH, _CH), jnp.int32),
        pltpu.VMEM((_NCH, _CH), jnp.int32),
        pltpu.VMEM((_CH, _H), jnp.bfloat16),
        pltpu.VMEM((_CH, _H), jnp.bfloat16),
        pltpu.VMEM_SHARED((_NA, _H), jnp.bfloat16),
        pltpu.SemaphoreType.DMA,
        pltpu.SemaphoreType.DMA,
    ]
    if with_deg:
        out_type.append(jax.ShapeDtypeStruct((_N,), jnp.float32))
        scratch += [
            pltpu.VMEM((_CH,), jnp.float32),
            pltpu.VMEM_SHARED((_NA,), jnp.float32),
            pltpu.VMEM((_STRIPE,), jnp.float32),
        ]
    return pl.kernel(
        functools.partial(_sc_agg_body, with_deg),
        out_type=out_type,
        mesh=mesh,
        scratch_types=scratch,
        compiler_params=pltpu.CompilerParams(use_tc_tiling_on_sc=False),
    )


_sc_agg_deg = _make_sc_agg(True)
_sc_agg = _make_sc_agg(False)


_RB = 2000
_GRID = _N // _RB


def _tc_layer0_body(x_ref, ws_ref, wn_ref, b_ref, zs_ref, zn_ref):
    x = x_ref[...]
    zs_ref[...] = (jnp.dot(x, ws_ref[...], preferred_element_type=jnp.float32)
                   + b_ref[...])
    zn = jnp.dot(x, wn_ref[...],
                 preferred_element_type=jnp.float32).astype(jnp.bfloat16)
    zn_ref[0] = zn[:, :_H]
    zn_ref[1] = zn[:, _H:]


def _tc_layer_body(zs_ref, a_ref, d_ref, ws_ref, wn_ref, b_ref,
                   ozs_ref, ozn_ref):
    deg = jnp.maximum(d_ref[...], 1.0)
    agg = jnp.concatenate([a_ref[0], a_ref[1]],
                          axis=1).astype(jnp.float32) / deg
    h = jnp.maximum(zs_ref[...] + agg, 0.0)
    ozs_ref[...] = (jnp.dot(h, ws_ref[...], preferred_element_type=jnp.float32)
                    + b_ref[...])
    zn = jnp.dot(h, wn_ref[...],
                 preferred_element_type=jnp.float32).astype(jnp.bfloat16)
    ozn_ref[0] = zn[:, :_H]
    ozn_ref[1] = zn[:, _H:]


def _tc_final_body(zs_ref, a_ref, d_ref, wf_ref, bf_ref, o_ref):
    i = pl.program_id(0)
    deg = jnp.maximum(d_ref[...], 1.0)
    agg = jnp.concatenate([a_ref[0], a_ref[1]],
                          axis=1).astype(jnp.float32) / deg
    h = jnp.maximum(zs_ref[...] + agg, 0.0)
    part = jnp.sum(h, axis=0, keepdims=True) * (1.0 / _N)
    part = jnp.dot(part, wf_ref[...], preferred_element_type=jnp.float32)

    @pl.when(i == 0)
    def _():
        o_ref[...] = bf_ref[...]

    o_ref[...] += part


_row_spec = pl.BlockSpec((_RB, _D), lambda i: (i, 0))
_a_spec = pl.BlockSpec((_NC, _RB, _H), lambda i: (0, i, 0))
_d_spec = pl.BlockSpec((_RB, 1), lambda i: (i, 0))
_w_spec = pl.BlockSpec((_D, _D), lambda i: (0, 0))
_b_spec = pl.BlockSpec((1, _D), lambda i: (0, 0))

_tc_layer0 = pl.pallas_call(
    _tc_layer0_body,
    grid=(_GRID,),
    in_specs=[_row_spec, _w_spec, _w_spec, _b_spec],
    out_specs=[_row_spec, _a_spec],
    out_shape=[jax.ShapeDtypeStruct((_N, _D), jnp.float32),
               jax.ShapeDtypeStruct((_NC, _N, _H), jnp.bfloat16)],
)

_tc_layer = pl.pallas_call(
    _tc_layer_body,
    grid=(_GRID,),
    in_specs=[_row_spec, _a_spec, _d_spec, _w_spec, _w_spec, _b_spec],
    out_specs=[_row_spec, _a_spec],
    out_shape=[jax.ShapeDtypeStruct((_N, _D), jnp.float32),
               jax.ShapeDtypeStruct((_NC, _N, _H), jnp.bfloat16)],
)

_tc_final = pl.pallas_call(
    _tc_final_body,
    grid=(_GRID,),
    in_specs=[_row_spec, _a_spec, _d_spec,
              pl.BlockSpec((_D, _C), lambda i: (0, 0)),
              pl.BlockSpec((1, _C), lambda i: (0, 0))],
    out_specs=pl.BlockSpec((1, _C), lambda i: (0, 0)),
    out_shape=jax.ShapeDtypeStruct((1, _C), jnp.float32),
)


def kernel(x, edge_index, ws0, wn0, b0, ws1, wn1, b1, ws2, wn2, b2, wf, bf):
    src = jnp.concatenate(
        [edge_index[0], jnp.zeros((_EPAD,), jnp.int32)]).reshape(
            _NS, _NCH, _CH)
    dst = jnp.concatenate(
        [edge_index[1], jnp.full((_EPAD,), _N, jnp.int32)]).reshape(
            _NS, _NCH, _CH)
    z2 = jnp.zeros((_NA, _H), jnp.bfloat16)

    zs0, zn0 = _tc_layer0(x, ws0, wn0, b0.reshape(1, _D))
    a0, dvec = _sc_agg_deg(src, dst, zn0, z2)
    d = dvec.reshape(_N, 1)
    zs1, zn1 = _tc_layer(zs0, a0, d, ws1, wn1, b1.reshape(1, _D))
    (a1,) = _sc_agg(src, dst, zn1, z2)
    zs2, zn2 = _tc_layer(zs1, a1, d, ws2, wn2, b2.reshape(1, _D))
    (a2,) = _sc_agg(src, dst, zn2, z2)
    out = _tc_final(zs2, a2, d, wf, bf.reshape(1, _C))
    return out
